# baseline (speedup 1.0000x reference)
import functools

import jax
import jax.numpy as jnp
from jax import lax
from jax.experimental import pallas as pl
from jax.experimental.pallas import tpu as pltpu
from jax.experimental.pallas import tpu_sc as plsc

N = 50000
NP = 50048
E = 800000
EP = 800768
EPW = EP // 32
CE = 4352
NB = EP // CE
CE2 = 1472
NB2 = EPW // CE2
F = 32

_MESH = dict(
    mesh=plsc.VectorSubcoreMesh(core_axis_name="c", subcore_axis_name="s"),
    compiler_params=pltpu.CompilerParams(needs_layout_passes=False),
)


def _wid():
    return lax.axis_index("s") * 2 + lax.axis_index("c")


def _vec_take(v, idx):
    return v.at[idx].get(mode="promise_in_bounds")


@functools.partial(
    pl.kernel,
    out_type=(jax.ShapeDtypeStruct((EP,), jnp.int32),
              jax.ShapeDtypeStruct((EP,), jnp.int32),
              jax.ShapeDtypeStruct((EP,), jnp.int32)),
    scratch_types=[pltpu.VMEM((EPW,), jnp.int32), pltpu.VMEM((EPW,), jnp.int32),
                   pltpu.VMEM((EPW,), jnp.int32), pltpu.VMEM((EPW,), jnp.int32),
                   pltpu.VMEM((EPW,), jnp.int32)],
    **_MESH,
)
def _p1_chunk_sort(src_hbm, dst_hbm, psrc_hbm, pdst_hbm, pe_hbm, sv, dv, osv,
                   odv, oev):
    wid = _wid()
    base = wid * EPW
    pltpu.sync_copy(src_hbm.at[pl.ds(base, EPW)], sv)
    pltpu.sync_copy(dst_hbm.at[pl.ds(base, EPW)], dv)
    io = lax.iota(jnp.int32, 16)

    def chunk(k, _):
        d16 = dv[pl.ds(k * 16, 16)]
        s16 = sv[pl.ds(k * 16, 16)]
        sd, ss = plsc.sort_key_val(d16, s16)
        scode = ss
        for j, sh in enumerate((1, 2, 4, 8)):
            prev = _vec_take(sd, jnp.maximum(io - sh, 0))
            m = (prev == sd) & (io >= sh)
            pidx = jnp.where(m, io - sh, io)
            scode = scode | (pidx << (16 + 4 * j))
        nxt = _vec_take(sd, jnp.minimum(io + 1, 15))
        last = (sd != nxt) | (io == 15)
        dcode = sd | (last.astype(jnp.int32) << 20)
        odv[pl.ds(k * 16, 16)] = dcode
        osv[pl.ds(k * 16, 16)] = scode
        oev[pl.ds(k * 16, 16)] = ss | (sd << 16)
        return 0

    lax.fori_loop(0, EPW // 16, chunk, 0, unroll=2)
    pltpu.sync_copy(osv, psrc_hbm.at[pl.ds(base, EPW)])
    pltpu.sync_copy(odv, pdst_hbm.at[pl.ds(base, EPW)])
    pltpu.sync_copy(oev, pe_hbm.at[pl.ds(base, EPW)])


@functools.partial(
    pl.kernel,
    out_type=jax.ShapeDtypeStruct((F, NP), jnp.float32),
    scratch_types=[pltpu.VMEM((EPW,), jnp.int32), pltpu.VMEM((NP,), jnp.float32)],
    **_MESH,
)
def _p2a_counts(pdst_hbm, cntp_hbm, dv, cv):
    wid = _wid()
    pltpu.sync_copy(pdst_hbm.at[pl.ds(wid * EPW, EPW)], dv)
    zero = jnp.zeros((16,), jnp.float32)

    def zb(i, _):
        cv[pl.ds(i * 16, 16)] = zero
        return 0

    lax.fori_loop(0, NP // 16, zb, 0, unroll=8)
    ones = jnp.full((16,), 1.0, jnp.float32)
    low = jnp.full((16,), 0xFFFF, jnp.int32)

    def chunk(k, _):
        d16 = dv[pl.ds(k * 16, 16)] & low
        plsc.addupdate_scatter(cv, [d16], ones)
        return 0

    lax.fori_loop(0, EPW // 16, chunk, 0, unroll=4)
    pltpu.sync_copy(cv, cntp_hbm.at[wid])


@functools.partial(
    pl.kernel,
    out_type=jax.ShapeDtypeStruct((F, NP), jnp.float32),
    scratch_types=[pltpu.VMEM((NP,), jnp.float32), pltpu.VMEM((NP,), jnp.float32),
                   pltpu.VMEM((CE2,), jnp.int32), pltpu.VMEM((CE2,), jnp.int32)],
    **_MESH,
)
def _p2b_agg0(x_hbm, psrc_hbm, pdst_hbm, aggp_hbm, xv, av, sb, db):
    wid = _wid()
    pltpu.sync_copy(x_hbm, xv)
    zero = jnp.zeros((16,), jnp.float32)

    def zb(i, _):
        av[pl.ds(i * 16, 16)] = zero
        return 0

    lax.fori_loop(0, NP // 16, zb, 0, unroll=8)
    low = jnp.full((16,), 0xFFFF, jnp.int32)

    def blk(b, _):
        base = wid * EPW + b * CE2
        pltpu.sync_copy(psrc_hbm.at[pl.ds(base, CE2)], sb)
        pltpu.sync_copy(pdst_hbm.at[pl.ds(base, CE2)], db)

        def chunk(k, _):
            s16 = sb[pl.ds(k * 16, 16)] & low
            d16 = db[pl.ds(k * 16, 16)] & low
            vals = plsc.load_gather(xv, [s16])
            plsc.addupdate_scatter(av, [d16], vals)
            return 0

        lax.fori_loop(0, CE2 // 16, chunk, 0, unroll=4)
        return 0

    lax.fori_loop(0, NB2, blk, 0)
    pltpu.sync_copy(av, aggp_hbm.at[wid])


@functools.partial(
    pl.kernel,
    out_type=jax.ShapeDtypeStruct((F, NP), jnp.float32),
    scratch_types=[pltpu.VMEM((NP,), jnp.float32), pltpu.VMEM((NP,), jnp.float32),
                   pltpu.VMEM((CE,), jnp.int32), pltpu.VMEM((CE,), jnp.int32),
                   pltpu.SemaphoreType.DMA, pltpu.SemaphoreType.DMA],
    **_MESH,
)
def _agg_sum(h_hbm, pe_hbm, out_hbm, hv, av, e0, e1, sem0, sem1):
    wid = _wid()
    pltpu.sync_copy(h_hbm.at[wid], hv)
    zero = jnp.zeros((16,), jnp.float32)

    def zb(i, _):
        av[pl.ds(i * 16, 16)] = zero
        return 0

    lax.fori_loop(0, NP // 16, zb, 0, unroll=8)
    low = jnp.full((16,), 0xFFFF, jnp.int32)
    sh16 = jnp.full((16,), 16, jnp.int32)

    def issue(b, eb, sem):
        pltpu.async_copy(pe_hbm.at[pl.ds(b * CE, CE)], eb, sem)

    def drain(eb, sem):
        pltpu.make_async_copy(pe_hbm.at[pl.ds(0, CE)], eb, sem).wait()

    NCH = CE // 16
    PF = 6

    def process(eb):
        def load(k):
            e = eb[pl.ds(k * 16, 16)]
            vals = plsc.load_gather(hv, [e & low])
            return lax.shift_right_logical(e, sh16), vals

        def commit(d16, vals):
            plsc.addupdate_scatter(av, [d16], vals)

        cs = tuple(load(k) for k in range(PF))

        def chunk(k, cs):
            commit(*cs[0])
            cn = load(k + PF)
            return (*cs[1:], cn)

        cs = lax.fori_loop(0, NCH - PF, chunk, cs, unroll=8)
        for c in cs:
            commit(*c)

    issue(0, e0, sem0)

    def blk_body(b, _):
        def even():
            drain(e0, sem0)
            pl.when(b + 1 < NB)(lambda: issue(b + 1, e1, sem1))
            process(e0)

        def odd():
            drain(e1, sem1)
            pl.when(b + 1 < NB)(lambda: issue(b + 1, e0, sem0))
            process(e1)

        lax.cond(b % 2 == 0, even, odd)
        return 0

    lax.fori_loop(0, NB, blk_body, 0)
    pltpu.sync_copy(av, out_hbm.at[wid])


def _make_agg(is_max):
    @functools.partial(
        pl.kernel,
        out_type=jax.ShapeDtypeStruct((F, NP), jnp.float32),
        scratch_types=[pltpu.VMEM((NP,), jnp.float32), pltpu.VMEM((NP,), jnp.float32),
                       pltpu.VMEM((CE,), jnp.int32), pltpu.VMEM((CE,), jnp.int32),
                       pltpu.VMEM((CE,), jnp.int32), pltpu.VMEM((CE,), jnp.int32),
                       pltpu.SemaphoreType.DMA, pltpu.SemaphoreType.DMA],
        **_MESH,
    )
    def agg_kernel(h_hbm, ps_hbm, pd_hbm, out_hbm, hv, av, ps0, pd0, ps1, pd1,
                   sem0, sem1):
        wid = _wid()
        pltpu.sync_copy(h_hbm.at[wid], hv)
        init = jnp.full((16,), float("-inf") if is_max else 0.0, jnp.float32)

        def zb(i, _):
            av[pl.ds(i * 16, 16)] = init
            return 0

        lax.fori_loop(0, NP // 16, zb, 0, unroll=8)

        low = jnp.full((16,), 0xFFFF, jnp.int32)
        fifteen = jnp.full((16,), 15, jnp.int32)
        shcon = [jnp.full((16,), 16 + 4 * j, jnp.int32) for j in range(4)]
        bit20 = jnp.full((16,), 1 << 20, jnp.int32)
        zero_i = jnp.zeros((16,), jnp.int32)

        def issue(b, psb, pdb, sem):
            pltpu.async_copy(ps_hbm.at[pl.ds(b * CE, CE)], psb, sem)
            pltpu.async_copy(pd_hbm.at[pl.ds(b * CE, CE)], pdb, sem)

        def drain(psb, pdb, sem):
            pltpu.make_async_copy(ps_hbm.at[pl.ds(0, CE)], psb, sem).wait()
            pltpu.make_async_copy(pd_hbm.at[pl.ds(0, CE)], pdb, sem).wait()

        NCH = CE // 16
        PF = 6

        def process(psb, pdb):
            def load(k):
                s = psb[pl.ds(k * 16, 16)]
                pdm = pdb[pl.ds(k * 16, 16)]
                vals = plsc.load_gather(hv, [s & low])
                if is_max:
                    v = vals
                    for j in range(4):
                        pidx = lax.shift_right_logical(s, shcon[j]) & fifteen
                        v = jnp.maximum(v, _vec_take(v, pidx))
                    return pdm, v
                return pdm, vals

            def commit(pdm, v):
                d16 = pdm & low
                if is_max:
                    last = (pdm & bit20) != zero_i
                    cur = plsc.load_gather(av, [d16])
                    plsc.store_scatter(av, [d16], jnp.maximum(cur, v), mask=last)
                else:
                    plsc.addupdate_scatter(av, [d16], v)

            cs = tuple(load(k) for k in range(PF))

            def chunk(k, cs):
                commit(*cs[0])
                cn = load(k + PF)
                return (*cs[1:], cn)

            cs = lax.fori_loop(0, NCH - PF, chunk, cs, unroll=8)
            for c in cs:
                commit(*c)

        issue(0, ps0, pd0, sem0)

        def blk_body(b, _):
            def even():
                drain(ps0, pd0, sem0)
                pl.when(b + 1 < NB)(lambda: issue(b + 1, ps1, pd1, sem1))
                process(ps0, pd0)

            def odd():
                drain(ps1, pd1, sem1)
                pl.when(b + 1 < NB)(lambda: issue(b + 1, ps0, pd0, sem0))
                process(ps1, pd1)

            lax.cond(b % 2 == 0, even, odd)
            return 0

        lax.fori_loop(0, NB, blk_body, 0)
        pltpu.sync_copy(av, out_hbm.at[wid])

    return agg_kernel


_agg_max = _make_agg(True)


def _tc_layer0(x2r, agg0p, cntp, wrel_t, wroot_t, b_t):
    def body(x_ref, a_ref, c_ref, wr_ref, wt_ref, b_ref, h_ref, r_ref):
        a0 = jnp.sum(a_ref[...], axis=0, keepdims=True)
        cnt = jnp.sum(c_ref[...], axis=0, keepdims=True) + 1.0
        x = x_ref[...]
        tot = a0 + x
        h = wr_ref[...] * tot + wt_ref[...] * x + b_ref[...]
        h_ref[...] = jnp.maximum(h, 0.0)
        r_ref[...] = 1.0 / cnt

    return pl.pallas_call(
        body,
        out_shape=(jax.ShapeDtypeStruct((F, NP), jnp.float32),
                   jax.ShapeDtypeStruct((1, NP), jnp.float32)),
    )(x2r, agg0p, cntp, wrel_t, wroot_t, b_t)


_DN = (((0,), (0,)), ((), ()))


def _tc_mid(h, agg, rcp, wrel, wroot, b_t, mode):
    def body(h_ref, a_ref, r_ref, wr_ref, wt_ref, b_ref, o_ref):
        hh = h_ref[...]
        ag = a_ref[...]
        if mode == "add":
            A = ag + hh
        elif mode == "mean":
            A = (ag + hh) * r_ref[...]
        else:
            A = jnp.maximum(ag, hh)
        y = (lax.dot_general(wr_ref[...], A, _DN, preferred_element_type=jnp.float32)
             + lax.dot_general(wt_ref[...], hh, _DN, preferred_element_type=jnp.float32)
             + b_ref[...])
        o_ref[...] = jnp.maximum(y, 0.0)

    return pl.pallas_call(
        body,
        out_shape=jax.ShapeDtypeStruct((F, NP), jnp.float32),
    )(h, agg, rcp, wrel, wroot, b_t)


def _tc_final(h, agg, wrel_out, wroot_out, b_out):
    def body(h_ref, a_ref, wr_ref, wt_ref, b_ref, o_ref):
        hh = h_ref[...]
        A = jnp.maximum(a_ref[...], hh)
        y = (lax.dot_general(wr_ref[...], A, _DN, preferred_element_type=jnp.float32)
             + lax.dot_general(wt_ref[...], hh, _DN, preferred_element_type=jnp.float32)
             + b_ref[...])
        o_ref[...] = jax.nn.sigmoid(y)

    return pl.pallas_call(
        body,
        out_shape=jax.ShapeDtypeStruct((1, NP), jnp.float32),
    )(h, agg, wrel_out, wroot_out, b_out)


def kernel(x, edge_index, Wrel_in, b_in, Wroot_in, Wrel_mid, b_mid, Wroot_mid,
           Wrel_out, b_out, Wroot_out):
    pad_e = EP - E
    src = jnp.concatenate([edge_index[0], jnp.zeros((pad_e,), jnp.int32)])
    dst = jnp.concatenate([edge_index[1], jnp.full((pad_e,), NP - 1, jnp.int32)])
    x2 = jnp.concatenate([x, jnp.zeros((NP - N,), jnp.float32)])
    x2r = x2.reshape(1, NP)

    psrc, pdst, pe = _p1_chunk_sort(src, dst)
    cntp = _p2a_counts(pdst)
    agg0p = _p2b_agg0(x2, psrc, pdst)

    h, rcp = _tc_layer0(x2r, agg0p, cntp,
                        Wrel_in.reshape(F, 1), Wroot_in.reshape(F, 1),
                        b_in.reshape(F, 1))

    aggrs = ("mean", "max", "add", "max", "mean", "max", "mean", "max", "mean", "max")
    for i, mode in enumerate(aggrs):
        agg = _agg_max(h, psrc, pdst) if mode == "max" else _agg_sum(h, pe)
        h = _tc_mid(h, agg, rcp, Wrel_mid[i], Wroot_mid[i],
                    b_mid[i].reshape(F, 1), mode)

    agg = _agg_max(h, psrc, pdst)
    out = _tc_final(h, agg, Wrel_out, Wroot_out, b_out.reshape(1, 1))
    return out.reshape(NP)[:N]


# speedup vs baseline: 1.1201x; 1.1201x over previous
import functools

import jax
import jax.numpy as jnp
from jax import lax
from jax.experimental import pallas as pl
from jax.experimental.pallas import tpu as pltpu
from jax.experimental.pallas import tpu_sc as plsc

N = 50000
NP = 50048
E = 800000
EP = 800768
EPW = EP // 32
CE = 4352
NB = EP // CE
CE2 = 1472
NB2 = EPW // CE2
F = 32

_MESH = dict(
    mesh=plsc.VectorSubcoreMesh(core_axis_name="c", subcore_axis_name="s"),
    compiler_params=pltpu.CompilerParams(needs_layout_passes=False),
)


def _wid():
    return lax.axis_index("s") * 2 + lax.axis_index("c")


def _vec_take(v, idx):
    return v.at[idx].get(mode="promise_in_bounds")


@functools.partial(
    pl.kernel,
    out_type=(jax.ShapeDtypeStruct((EP,), jnp.int32),
              jax.ShapeDtypeStruct((EP,), jnp.int32),
              jax.ShapeDtypeStruct((EP,), jnp.int32)),
    scratch_types=[pltpu.VMEM((EPW,), jnp.int32), pltpu.VMEM((EPW,), jnp.int32),
                   pltpu.VMEM((EPW,), jnp.int32), pltpu.VMEM((EPW,), jnp.int32),
                   pltpu.VMEM((EPW,), jnp.int32)],
    **_MESH,
)
def _p1_chunk_sort(src_hbm, dst_hbm, psrc_hbm, pdst_hbm, pe_hbm, sv, dv, osv,
                   odv, oev):
    wid = _wid()
    base = wid * EPW
    pltpu.sync_copy(src_hbm.at[pl.ds(base, EPW)], sv)
    pltpu.sync_copy(dst_hbm.at[pl.ds(base, EPW)], dv)
    io = lax.iota(jnp.int32, 16)

    def chunk(k, _):
        d16 = dv[pl.ds(k * 16, 16)]
        s16 = sv[pl.ds(k * 16, 16)]
        sd, ss = plsc.sort_key_val(d16, s16)
        scode = ss
        for j, sh in enumerate((1, 2, 4, 8)):
            prev = _vec_take(sd, jnp.maximum(io - sh, 0))
            m = (prev == sd) & (io >= sh)
            pidx = jnp.where(m, io - sh, io)
            scode = scode | (pidx << (16 + 4 * j))
        nxt = _vec_take(sd, jnp.minimum(io + 1, 15))
        last = (sd != nxt) | (io == 15)
        dcode = sd | (last.astype(jnp.int32) << 20)
        odv[pl.ds(k * 16, 16)] = dcode
        osv[pl.ds(k * 16, 16)] = scode
        oev[pl.ds(k * 16, 16)] = ss | (sd << 16)
        return 0

    lax.fori_loop(0, EPW // 16, chunk, 0, unroll=2)
    pltpu.sync_copy(osv, psrc_hbm.at[pl.ds(base, EPW)])
    pltpu.sync_copy(odv, pdst_hbm.at[pl.ds(base, EPW)])
    pltpu.sync_copy(oev, pe_hbm.at[pl.ds(base, EPW)])


@functools.partial(
    pl.kernel,
    out_type=jax.ShapeDtypeStruct((F, NP), jnp.float32),
    scratch_types=[pltpu.VMEM((EPW,), jnp.int32), pltpu.VMEM((NP,), jnp.float32)],
    **_MESH,
)
def _p2a_counts(pdst_hbm, cntp_hbm, dv, cv):
    wid = _wid()
    pltpu.sync_copy(pdst_hbm.at[pl.ds(wid * EPW, EPW)], dv)
    zero = jnp.zeros((16,), jnp.float32)

    def zb(i, _):
        cv[pl.ds(i * 16, 16)] = zero
        return 0

    lax.fori_loop(0, NP // 16, zb, 0, unroll=8)
    ones = jnp.full((16,), 1.0, jnp.float32)
    low = jnp.full((16,), 0xFFFF, jnp.int32)

    def chunk(k, _):
        d16 = dv[pl.ds(k * 16, 16)] & low
        plsc.addupdate_scatter(cv, [d16], ones)
        return 0

    lax.fori_loop(0, EPW // 16, chunk, 0, unroll=4)
    pltpu.sync_copy(cv, cntp_hbm.at[wid])


@functools.partial(
    pl.kernel,
    out_type=jax.ShapeDtypeStruct((F, NP), jnp.float32),
    scratch_types=[pltpu.VMEM((NP,), jnp.float32), pltpu.VMEM((NP,), jnp.float32),
                   pltpu.VMEM((CE2,), jnp.int32), pltpu.VMEM((CE2,), jnp.int32)],
    **_MESH,
)
def _p2b_agg0(x_hbm, psrc_hbm, pdst_hbm, aggp_hbm, xv, av, sb, db):
    wid = _wid()
    pltpu.sync_copy(x_hbm, xv)
    zero = jnp.zeros((16,), jnp.float32)

    def zb(i, _):
        av[pl.ds(i * 16, 16)] = zero
        return 0

    lax.fori_loop(0, NP // 16, zb, 0, unroll=8)
    low = jnp.full((16,), 0xFFFF, jnp.int32)

    def blk(b, _):
        base = wid * EPW + b * CE2
        pltpu.sync_copy(psrc_hbm.at[pl.ds(base, CE2)], sb)
        pltpu.sync_copy(pdst_hbm.at[pl.ds(base, CE2)], db)

        def chunk(k, _):
            s16 = sb[pl.ds(k * 16, 16)] & low
            d16 = db[pl.ds(k * 16, 16)] & low
            vals = plsc.load_gather(xv, [s16])
            plsc.addupdate_scatter(av, [d16], vals)
            return 0

        lax.fori_loop(0, CE2 // 16, chunk, 0, unroll=4)
        return 0

    lax.fori_loop(0, NB2, blk, 0)
    pltpu.sync_copy(av, aggp_hbm.at[wid])


@functools.partial(
    pl.kernel,
    out_type=jax.ShapeDtypeStruct((F, NP), jnp.float32),
    scratch_types=[pltpu.VMEM((NP,), jnp.float32), pltpu.VMEM((NP,), jnp.float32),
                   pltpu.VMEM((CE,), jnp.int32), pltpu.VMEM((CE,), jnp.int32),
                   pltpu.SemaphoreType.DMA, pltpu.SemaphoreType.DMA],
    **_MESH,
)
def _agg_sum(h_hbm, pe_hbm, out_hbm, hv, av, e0, e1, sem0, sem1):
    wid = _wid()
    pltpu.sync_copy(h_hbm.at[wid], hv)
    zero = jnp.zeros((16,), jnp.float32)

    def zb(i, _):
        av[pl.ds(i * 16, 16)] = zero
        return 0

    lax.fori_loop(0, NP // 16, zb, 0, unroll=8)
    low = jnp.full((16,), 0xFFFF, jnp.int32)
    sh16 = jnp.full((16,), 16, jnp.int32)

    def issue(b, eb, sem):
        pltpu.async_copy(pe_hbm.at[pl.ds(b * CE, CE)], eb, sem)

    def drain(eb, sem):
        pltpu.make_async_copy(pe_hbm.at[pl.ds(0, CE)], eb, sem).wait()

    NCH = CE // 16
    PF = 4

    def process(eb):
        def load(k):
            e = eb[pl.ds(k * 16, 16)]
            vals = plsc.load_gather(hv, [e & low])
            return lax.shift_right_logical(e, sh16), vals

        def commit(d16, vals):
            plsc.addupdate_scatter(av, [d16], vals)

        cs = tuple(load(k) for k in range(PF))

        def chunk(k, cs):
            commit(*cs[0])
            cn = load(k + PF)
            return (*cs[1:], cn)

        cs = lax.fori_loop(0, NCH - PF, chunk, cs, unroll=4)
        for c in cs:
            commit(*c)

    issue(0, e0, sem0)

    def blk_body(b, _):
        def even():
            drain(e0, sem0)
            pl.when(b + 1 < NB)(lambda: issue(b + 1, e1, sem1))
            process(e0)

        def odd():
            drain(e1, sem1)
            pl.when(b + 1 < NB)(lambda: issue(b + 1, e0, sem0))
            process(e1)

        lax.cond(b % 2 == 0, even, odd)
        return 0

    lax.fori_loop(0, NB, blk_body, 0)
    pltpu.sync_copy(av, out_hbm.at[wid])


def _make_agg(is_max):
    @functools.partial(
        pl.kernel,
        out_type=jax.ShapeDtypeStruct((F, NP), jnp.float32),
        scratch_types=[pltpu.VMEM((NP,), jnp.float32), pltpu.VMEM((NP,), jnp.float32),
                       pltpu.VMEM((CE,), jnp.int32), pltpu.VMEM((CE,), jnp.int32),
                       pltpu.VMEM((CE,), jnp.int32), pltpu.VMEM((CE,), jnp.int32),
                       pltpu.SemaphoreType.DMA, pltpu.SemaphoreType.DMA],
        **_MESH,
    )
    def agg_kernel(h_hbm, ps_hbm, pd_hbm, out_hbm, hv, av, ps0, pd0, ps1, pd1,
                   sem0, sem1):
        wid = _wid()
        pltpu.sync_copy(h_hbm.at[wid], hv)
        init = jnp.full((16,), float("-inf") if is_max else 0.0, jnp.float32)

        def zb(i, _):
            av[pl.ds(i * 16, 16)] = init
            return 0

        lax.fori_loop(0, NP // 16, zb, 0, unroll=8)

        low = jnp.full((16,), 0xFFFF, jnp.int32)
        fifteen = jnp.full((16,), 15, jnp.int32)
        shcon = [jnp.full((16,), 16 + 4 * j, jnp.int32) for j in range(4)]
        bit20 = jnp.full((16,), 1 << 20, jnp.int32)
        zero_i = jnp.zeros((16,), jnp.int32)

        def issue(b, psb, pdb, sem):
            pltpu.async_copy(ps_hbm.at[pl.ds(b * CE, CE)], psb, sem)
            pltpu.async_copy(pd_hbm.at[pl.ds(b * CE, CE)], pdb, sem)

        def drain(psb, pdb, sem):
            pltpu.make_async_copy(ps_hbm.at[pl.ds(0, CE)], psb, sem).wait()
            pltpu.make_async_copy(pd_hbm.at[pl.ds(0, CE)], pdb, sem).wait()

        NCH = CE // 16
        PF = 4

        def process(psb, pdb):
            def load(k):
                s = psb[pl.ds(k * 16, 16)]
                pdm = pdb[pl.ds(k * 16, 16)]
                vals = plsc.load_gather(hv, [s & low])
                if is_max:
                    v = vals
                    for j in range(4):
                        pidx = lax.shift_right_logical(s, shcon[j]) & fifteen
                        v = jnp.maximum(v, _vec_take(v, pidx))
                    return pdm, v
                return pdm, vals

            def commit(pdm, v):
                d16 = pdm & low
                if is_max:
                    last = (pdm & bit20) != zero_i
                    cur = plsc.load_gather(av, [d16])
                    plsc.store_scatter(av, [d16], jnp.maximum(cur, v), mask=last)
                else:
                    plsc.addupdate_scatter(av, [d16], v)

            cs = tuple(load(k) for k in range(PF))

            def chunk(k, cs):
                commit(*cs[0])
                cn = load(k + PF)
                return (*cs[1:], cn)

            cs = lax.fori_loop(0, NCH - PF, chunk, cs, unroll=4)
            for c in cs:
                commit(*c)

        issue(0, ps0, pd0, sem0)

        def blk_body(b, _):
            def even():
                drain(ps0, pd0, sem0)
                pl.when(b + 1 < NB)(lambda: issue(b + 1, ps1, pd1, sem1))
                process(ps0, pd0)

            def odd():
                drain(ps1, pd1, sem1)
                pl.when(b + 1 < NB)(lambda: issue(b + 1, ps0, pd0, sem0))
                process(ps1, pd1)

            lax.cond(b % 2 == 0, even, odd)
            return 0

        lax.fori_loop(0, NB, blk_body, 0)
        pltpu.sync_copy(av, out_hbm.at[wid])

    return agg_kernel


_agg_max = _make_agg(True)


def _tc_layer0(x2r, agg0p, cntp, wrel_t, wroot_t, b_t):
    def body(x_ref, a_ref, c_ref, wr_ref, wt_ref, b_ref, h_ref, r_ref):
        a0 = jnp.sum(a_ref[...], axis=0, keepdims=True)
        cnt = jnp.sum(c_ref[...], axis=0, keepdims=True) + 1.0
        x = x_ref[...]
        tot = a0 + x
        h = wr_ref[...] * tot + wt_ref[...] * x + b_ref[...]
        h_ref[...] = jnp.maximum(h, 0.0)
        r_ref[...] = 1.0 / cnt

    return pl.pallas_call(
        body,
        out_shape=(jax.ShapeDtypeStruct((F, NP), jnp.float32),
                   jax.ShapeDtypeStruct((1, NP), jnp.float32)),
    )(x2r, agg0p, cntp, wrel_t, wroot_t, b_t)


_DN = (((0,), (0,)), ((), ()))


def _tc_mid(h, agg, rcp, wrel, wroot, b_t, mode):
    def body(h_ref, a_ref, r_ref, wr_ref, wt_ref, b_ref, o_ref):
        hh = h_ref[...]
        ag = a_ref[...]
        if mode == "add":
            A = ag + hh
        elif mode == "mean":
            A = (ag + hh) * r_ref[...]
        else:
            A = jnp.maximum(ag, hh)
        y = (lax.dot_general(wr_ref[...], A, _DN, preferred_element_type=jnp.float32)
             + lax.dot_general(wt_ref[...], hh, _DN, preferred_element_type=jnp.float32)
             + b_ref[...])
        o_ref[...] = jnp.maximum(y, 0.0)

    return pl.pallas_call(
        body,
        out_shape=jax.ShapeDtypeStruct((F, NP), jnp.float32),
    )(h, agg, rcp, wrel, wroot, b_t)


def _tc_final(h, agg, wrel_out, wroot_out, b_out):
    def body(h_ref, a_ref, wr_ref, wt_ref, b_ref, o_ref):
        hh = h_ref[...]
        A = jnp.maximum(a_ref[...], hh)
        y = (lax.dot_general(wr_ref[...], A, _DN, preferred_element_type=jnp.float32)
             + lax.dot_general(wt_ref[...], hh, _DN, preferred_element_type=jnp.float32)
             + b_ref[...])
        o_ref[...] = jax.nn.sigmoid(y)

    return pl.pallas_call(
        body,
        out_shape=jax.ShapeDtypeStruct((1, NP), jnp.float32),
    )(h, agg, wrel_out, wroot_out, b_out)


def kernel(x, edge_index, Wrel_in, b_in, Wroot_in, Wrel_mid, b_mid, Wroot_mid,
           Wrel_out, b_out, Wroot_out):
    pad_e = EP - E
    src = jnp.concatenate([edge_index[0], jnp.zeros((pad_e,), jnp.int32)])
    dst = jnp.concatenate([edge_index[1], jnp.full((pad_e,), NP - 1, jnp.int32)])
    x2 = jnp.concatenate([x, jnp.zeros((NP - N,), jnp.float32)])
    x2r = x2.reshape(1, NP)

    psrc, pdst, pe = _p1_chunk_sort(src, dst)
    cntp = _p2a_counts(pdst)
    agg0p = _p2b_agg0(x2, psrc, pdst)

    h, rcp = _tc_layer0(x2r, agg0p, cntp,
                        Wrel_in.reshape(F, 1), Wroot_in.reshape(F, 1),
                        b_in.reshape(F, 1))

    aggrs = ("mean", "max", "add", "max", "mean", "max", "mean", "max", "mean", "max")
    for i, mode in enumerate(aggrs):
        agg = _agg_max(h, psrc, pdst) if mode == "max" else _agg_sum(h, pe)
        h = _tc_mid(h, agg, rcp, Wrel_mid[i], Wroot_mid[i],
                    b_mid[i].reshape(F, 1), mode)

    agg = _agg_max(h, psrc, pdst)
    out = _tc_final(h, agg, Wrel_out, Wroot_out, b_out.reshape(1, 1))
    return out.reshape(NP)[:N]
